# baseline (speedup 1.0000x reference)
import functools

import jax
import jax.numpy as jnp
from jax import lax
from jax.experimental import pallas as pl
from jax.experimental.pallas import tpu as pltpu
from jax.experimental.pallas import tpu_sc as plsc

_NUM_ITEMS = 100000
_EMBED_DIM = 64
_FROZEN_DIM = 128
_NUM_TAGS = 1000
_TAG_DIM = 16
_TAGS_PER_ITEM = 5
_TAG_PACK = 2

_NC = 2
_NS = 16
_NW = _NC * _NS

_B = 4096
_H = 50
_C = 128
_NBT = _B // _C
_NSLAB = _H * _NBT
_PER_W = _NSLAB // _NW
_NSTEP = _PER_W // 2
_OUT_DIM = _FROZEN_DIM + _EMBED_DIM + _TAG_DIM
_NDT = _OUT_DIM // 8
_PITCH = _C + 1
_L = 16


def _sc_body(ids_hbm, ids4_hbm, frozen_hbm, tags_hbm, item_w_hbm, tag_w_hbm,
             w_hbm,
             idx0, idx1, idx4_0, idx4_1, frozen0, frozen1, learn0, learn1,
             tags0, tags1, slab0, slab1, tag_tab, gsem0, gsem1, wsem0, wsem1,
             isem0, isem1):
    wid = lax.axis_index("s") * _NC + lax.axis_index("c")
    base = wid * _PER_W

    idx_c = (idx0, idx1)
    idx4_c = (idx4_0, idx4_1)
    frozen_c = (frozen0, frozen1)
    learn_c = (learn0, learn1)
    tags_c = (tags0, tags1)
    slab_c = (slab0, slab1)
    gsem = (gsem0, gsem1)
    wsem = (wsem0, wsem1)
    isem = (isem0, isem1)

    pltpu.sync_copy(tag_w_hbm, tag_tab)

    iota = lax.iota(jnp.int32, _L)
    inv_tags = jnp.float32(1.0 / _TAGS_PER_ITEM)

    dtv = []
    drv = []
    for k in range(12):
        d = k * _L + iota
        dtv.append(lax.shift_right_logical(d, 3))
        drv.append(lax.bitwise_and(d, 7))

    def idx_start(g, b):
        pltpu.async_copy(ids_hbm.at[base + g], idx_c[b], isem[b])
        pltpu.async_copy(ids4_hbm.at[base + g], idx4_c[b], isem[b])

    def ig(g, b):
        pltpu.make_async_copy(ids_hbm.at[base], idx_c[b], isem[b]).wait()
        pltpu.make_async_copy(ids4_hbm.at[base], idx4_c[b], isem[b]).wait()
        idxr = idx_c[b]
        pltpu.async_copy(frozen_hbm.at[idxr], frozen_c[b], gsem[b])
        pltpu.async_copy(item_w_hbm.at[idxr], learn_c[b], gsem[b])
        pltpu.async_copy(tags_hbm.at[idx4_c[b]], tags_c[b], gsem[b])

    def waitg(b):
        idxr = idx_c[b]
        pltpu.make_async_copy(frozen_hbm.at[idxr], frozen_c[b], gsem[b]).wait()
        pltpu.make_async_copy(item_w_hbm.at[idxr], learn_c[b], gsem[b]).wait()
        pltpu.make_async_copy(tags_hbm.at[idxr], tags_c[b], gsem[b]).wait()

    def iw(g, b):
        s = base + g
        l = s // _NBT
        bt = s % _NBT
        pltpu.async_copy(slab_c[b].at[:, :, pl.ds(0, _C)],
                         w_hbm.at[l, :, bt], wsem[b])

    def waitw(b):
        pltpu.make_async_copy(slab_c[b].at[:, :, pl.ds(0, _C)],
                              w_hbm.at[0, :, 0], wsem[b]).wait()

    def compute(b):
        fz, ln, tg, W = frozen_c[b], learn_c[b], tags_c[b], slab_c[b]
        ids_v = idx_c[b]

        @plsc.parallel_loop(0, _C, 1, unroll=4)
        def _item(br):
            col = jnp.full((_L,), 0, jnp.int32) + br
            for k in range(8):
                v = fz[br, pl.ds(k * _L, _L)]
                plsc.store_scatter(W, [dtv[k], drv[k], col], v)
            for k in range(4):
                v = ln[br, pl.ds(k * _L, _L)]
                plsc.store_scatter(W, [dtv[8 + k], drv[8 + k], col], v)

        @plsc.parallel_loop(0, _C // _L, 1, unroll=2)
        def _grp(h):
            rows = h * _L + iota
            ids16 = ids_v[pl.ds(h * _L, _L)]
            col0 = lax.shift_left(lax.bitwise_and(ids16, 3), 1)
            w0 = plsc.load_gather(tg, [rows, col0])
            w1 = plsc.load_gather(tg, [rows, col0 + 1])
            mask10 = jnp.full((_L,), 1023, jnp.int32)
            tv = [
                lax.bitwise_and(w0, mask10),
                lax.bitwise_and(lax.shift_right_logical(w0, 10), mask10),
                lax.shift_right_logical(w0, 20),
                lax.bitwise_and(w1, mask10),
                lax.shift_right_logical(w1, 10),
            ]
            for d in range(_TAG_DIM):
                dcol = jnp.full((_L,), d, jnp.int32)
                acc = plsc.load_gather(tag_tab, [tv[0], dcol])
                for t in range(1, _TAGS_PER_ITEM):
                    acc = acc + plsc.load_gather(tag_tab, [tv[t], dcol])
                dd = _FROZEN_DIM + _EMBED_DIM + d
                W[dd >> 3, dd & 7, pl.ds(h * _L, _L)] = acc * inv_tags

    idx_start(0, 0)
    idx_start(1, 1)
    ig(0, 0)

    def step(s, carry):
        g0 = 2 * s
        pl.when(s > 0)(lambda: waitw(1))
        ig(g0 + 1, 1)
        waitg(0)
        compute(0)
        pl.when(s < _NSTEP - 1)(lambda: idx_start(g0 + 2, 0))
        iw(g0, 0)

        def prefetch_even():
            waitw(0)
            ig(g0 + 2, 0)

        pl.when(s < _NSTEP - 1)(prefetch_even)
        waitg(1)
        compute(1)
        pl.when(s < _NSTEP - 1)(lambda: idx_start(g0 + 3, 1))
        iw(g0 + 1, 1)
        return carry

    lax.fori_loop(0, _NSTEP, step, 0)
    waitw(0)
    waitw(1)


_sc_kernel = functools.partial(
    pl.kernel,
    out_type=jax.ShapeDtypeStruct((_H, _NDT, _NBT, 8, _C), jnp.float32),
    mesh=plsc.VectorSubcoreMesh(core_axis_name="c", subcore_axis_name="s",
                                num_cores=_NC, num_subcores=_NS),
    scratch_types=[
        pltpu.VMEM((_C,), jnp.int32),
        pltpu.VMEM((_C,), jnp.int32),
        pltpu.VMEM((_C,), jnp.int32),
        pltpu.VMEM((_C,), jnp.int32),
        pltpu.VMEM((_C, _FROZEN_DIM), jnp.float32),
        pltpu.VMEM((_C, _FROZEN_DIM), jnp.float32),
        pltpu.VMEM((_C, _EMBED_DIM), jnp.float32),
        pltpu.VMEM((_C, _EMBED_DIM), jnp.float32),
        pltpu.VMEM((_C, 8), jnp.int32),
        pltpu.VMEM((_C, 8), jnp.int32),
        pltpu.VMEM((_NDT, 8, _PITCH), jnp.float32),
        pltpu.VMEM((_NDT, 8, _PITCH), jnp.float32),
        pltpu.VMEM((_NUM_TAGS, _TAG_DIM), jnp.float32),
        pltpu.SemaphoreType.DMA,
        pltpu.SemaphoreType.DMA,
        pltpu.SemaphoreType.DMA,
        pltpu.SemaphoreType.DMA,
        pltpu.SemaphoreType.DMA,
        pltpu.SemaphoreType.DMA,
    ],
    compiler_params=pltpu.CompilerParams(use_tc_tiling_on_sc=False,
                                         needs_layout_passes=False),
)(_sc_body)


def kernel(item_ids, frozen_emb, item_tags, item_emb_w, tag_emb_w):
    ids_t = item_ids.astype(jnp.int32).T.reshape(_NSLAB, _C)
    t = item_tags.astype(jnp.int32)
    w0 = t[:, 0] | (t[:, 1] << 10) | (t[:, 2] << 20)
    w1 = t[:, 3] | (t[:, 4] << 10)
    flat = jnp.zeros((2 * _NUM_ITEMS + 192,), jnp.int32)
    flat = flat.at[0:2 * _NUM_ITEMS:2].set(w0)
    flat = flat.at[1:2 * _NUM_ITEMS:2].set(w1)
    packed = flat.reshape((_NUM_ITEMS + 96) // 4, 8)
    ids4 = ids_t >> 2
    w = _sc_kernel(ids_t, ids4, frozen_emb, packed, item_emb_w, tag_emb_w)
    return jnp.transpose(w, (2, 4, 0, 1, 3)).reshape(_B, _H, _OUT_DIM)


# speedup vs baseline: 3.7182x; 3.7182x over previous
import functools

import jax
import jax.numpy as jnp
from jax import lax
from jax.experimental import pallas as pl
from jax.experimental.pallas import tpu as pltpu
from jax.experimental.pallas import tpu_sc as plsc

_NUM_ITEMS = 100000
_EMBED_DIM = 64
_FROZEN_DIM = 128
_NUM_TAGS = 1000
_TAG_DIM = 16
_TAGS_PER_ITEM = 5
_TAG_PACK = 2

_NC = 2
_NS = 16
_NW = _NC * _NS

_B = 4096
_H = 50
_C = 128
_NBT = _B // _C
_NSLAB = _H * _NBT
_PER_W = _NSLAB // _NW
_NSTEP = _PER_W // 2
_OUT_DIM = _FROZEN_DIM + _EMBED_DIM + _TAG_DIM
_NDT = _OUT_DIM // 8
_PITCH = _C + 1
_L = 16


def _sc_body(ids_hbm, ids8_hbm, frozen_hbm, tags0_hbm, tags1_hbm, item_w_hbm,
             tag_w_hbm, w_hbm,
             idx0, idx1, idx8_0, idx8_1, frozen0, frozen1, learn0, learn1,
             tags0a, tags0b, tags1a, tags1b, slab0, slab1, tag_tab,
             gsem0, gsem1, wsem0, wsem1, isem0, isem1):
    wid = lax.axis_index("s") * _NC + lax.axis_index("c")
    base = wid * _PER_W

    idx_c = (idx0, idx1)
    idx8_c = (idx8_0, idx8_1)
    frozen_c = (frozen0, frozen1)
    learn_c = (learn0, learn1)
    tags0_c = (tags0a, tags0b)
    tags1_c = (tags1a, tags1b)
    slab_c = (slab0, slab1)
    gsem = (gsem0, gsem1)
    wsem = (wsem0, wsem1)
    isem = (isem0, isem1)

    pltpu.sync_copy(tag_w_hbm, tag_tab)

    iota = lax.iota(jnp.int32, _L)
    inv_tags = jnp.float32(1.0 / _TAGS_PER_ITEM)

    dtv = []
    drv = []
    for k in range(12):
        d = k * _L + iota
        dtv.append(lax.shift_right_logical(d, 3))
        drv.append(lax.bitwise_and(d, 7))

    def idx_start(g, b):
        pltpu.async_copy(ids_hbm.at[base + g], idx_c[b], isem[b])
        pltpu.async_copy(ids8_hbm.at[base + g], idx8_c[b], isem[b])

    def ig(g, b):
        pltpu.make_async_copy(ids_hbm.at[base], idx_c[b], isem[b]).wait()
        pltpu.make_async_copy(ids8_hbm.at[base], idx8_c[b], isem[b]).wait()
        idxr = idx_c[b]
        pltpu.async_copy(frozen_hbm.at[idxr], frozen_c[b], gsem[b])
        pltpu.async_copy(item_w_hbm.at[idxr], learn_c[b], gsem[b])
        pltpu.async_copy(tags0_hbm.at[idx8_c[b]], tags0_c[b], gsem[b])
        pltpu.async_copy(tags1_hbm.at[idx8_c[b]], tags1_c[b], gsem[b])

    def waitg(b):
        idxr = idx_c[b]
        pltpu.make_async_copy(frozen_hbm.at[idxr], frozen_c[b], gsem[b]).wait()
        pltpu.make_async_copy(item_w_hbm.at[idxr], learn_c[b], gsem[b]).wait()
        pltpu.make_async_copy(tags0_hbm.at[idx8_c[b]], tags0_c[b], gsem[b]).wait()
        pltpu.make_async_copy(tags1_hbm.at[idx8_c[b]], tags1_c[b], gsem[b]).wait()

    def iw(g, b):
        s = base + g
        l = s // _NBT
        bt = s % _NBT
        pltpu.async_copy(slab_c[b].at[:, :, pl.ds(0, _C)],
                         w_hbm.at[l, :, bt], wsem[b])

    def waitw(b):
        pltpu.make_async_copy(slab_c[b].at[:, :, pl.ds(0, _C)],
                              w_hbm.at[0, :, 0], wsem[b]).wait()

    def compute(b):
        fz, ln, W = frozen_c[b], learn_c[b], slab_c[b]
        tg0, tg1 = tags0_c[b], tags1_c[b]
        ids_v = idx_c[b]

        @plsc.parallel_loop(0, _C, 1, unroll=4)
        def _item(br):
            col = jnp.full((_L,), 0, jnp.int32) + br
            for k in range(8):
                v = fz[br, pl.ds(k * _L, _L)]
                plsc.store_scatter(W, [dtv[k], drv[k], col], v)
            for k in range(4):
                v = ln[br, pl.ds(k * _L, _L)]
                plsc.store_scatter(W, [dtv[8 + k], drv[8 + k], col], v)

        @plsc.parallel_loop(0, _C // _L, 1, unroll=2)
        def _grp(h):
            rows = h * _L + iota
            ids16 = ids_v[pl.ds(h * _L, _L)]
            col = lax.bitwise_and(ids16, 7)
            w0 = plsc.load_gather(tg0, [rows, col])
            w1 = plsc.load_gather(tg1, [rows, col])
            mask10 = jnp.full((_L,), 1023, jnp.int32)
            tv = [
                lax.bitwise_and(w0, mask10),
                lax.bitwise_and(lax.shift_right_logical(w0, 10), mask10),
                lax.shift_right_logical(w0, 20),
                lax.bitwise_and(w1, mask10),
                lax.shift_right_logical(w1, 10),
            ]
            for d in range(_TAG_DIM):
                dcol = jnp.full((_L,), d, jnp.int32)
                acc = plsc.load_gather(tag_tab, [tv[0], dcol])
                for t in range(1, _TAGS_PER_ITEM):
                    acc = acc + plsc.load_gather(tag_tab, [tv[t], dcol])
                dd = _FROZEN_DIM + _EMBED_DIM + d
                W[dd >> 3, dd & 7, pl.ds(h * _L, _L)] = acc * inv_tags

    idx_start(0, 0)
    idx_start(1, 1)
    ig(0, 0)

    def step(s, carry):
        g0 = 2 * s
        pl.when(s > 0)(lambda: waitw(1))
        ig(g0 + 1, 1)
        waitg(0)
        compute(0)
        pl.when(s < _NSTEP - 1)(lambda: idx_start(g0 + 2, 0))
        iw(g0, 0)

        def prefetch_even():
            waitw(0)
            ig(g0 + 2, 0)

        pl.when(s < _NSTEP - 1)(prefetch_even)
        waitg(1)
        compute(1)
        pl.when(s < _NSTEP - 1)(lambda: idx_start(g0 + 3, 1))
        iw(g0 + 1, 1)
        return carry

    lax.fori_loop(0, _NSTEP, step, 0)
    waitw(0)
    waitw(1)


_sc_kernel = functools.partial(
    pl.kernel,
    out_type=jax.ShapeDtypeStruct((_H, _NDT, _NBT, 8, _C), jnp.float32),
    mesh=plsc.VectorSubcoreMesh(core_axis_name="c", subcore_axis_name="s",
                                num_cores=_NC, num_subcores=_NS),
    scratch_types=[
        pltpu.VMEM((_C,), jnp.int32),
        pltpu.VMEM((_C,), jnp.int32),
        pltpu.VMEM((_C,), jnp.int32),
        pltpu.VMEM((_C,), jnp.int32),
        pltpu.VMEM((_C, _FROZEN_DIM), jnp.float32),
        pltpu.VMEM((_C, _FROZEN_DIM), jnp.float32),
        pltpu.VMEM((_C, _EMBED_DIM), jnp.float32),
        pltpu.VMEM((_C, _EMBED_DIM), jnp.float32),
        pltpu.VMEM((_C, 8), jnp.int32),
        pltpu.VMEM((_C, 8), jnp.int32),
        pltpu.VMEM((_C, 8), jnp.int32),
        pltpu.VMEM((_C, 8), jnp.int32),
        pltpu.VMEM((_NDT, 8, _PITCH), jnp.float32),
        pltpu.VMEM((_NDT, 8, _PITCH), jnp.float32),
        pltpu.VMEM((_NUM_TAGS, _TAG_DIM), jnp.float32),
        pltpu.SemaphoreType.DMA,
        pltpu.SemaphoreType.DMA,
        pltpu.SemaphoreType.DMA,
        pltpu.SemaphoreType.DMA,
        pltpu.SemaphoreType.DMA,
        pltpu.SemaphoreType.DMA,
    ],
    compiler_params=pltpu.CompilerParams(use_tc_tiling_on_sc=False,
                                         needs_layout_passes=False),
)(_sc_body)


def kernel(item_ids, frozen_emb, item_tags, item_emb_w, tag_emb_w):
    ids_t = item_ids.astype(jnp.int32).T.reshape(_NSLAB, _C)
    t = item_tags.astype(jnp.int32)
    w0 = t[:, 0] | (t[:, 1] << 10) | (t[:, 2] << 20)
    w1 = t[:, 3] | (t[:, 4] << 10)
    w0t = w0.reshape(_NUM_ITEMS // 8, 8)
    w1t = w1.reshape(_NUM_ITEMS // 8, 8)
    ids8 = ids_t >> 3
    w = _sc_kernel(ids_t, ids8, frozen_emb, w0t, w1t, item_emb_w, tag_emb_w)
    return jnp.transpose(w, (2, 4, 0, 1, 3)).reshape(_B, _H, _OUT_DIM)


# speedup vs baseline: 3.7981x; 1.0215x over previous
import functools

import jax
import jax.numpy as jnp
from jax import lax
from jax.experimental import pallas as pl
from jax.experimental.pallas import tpu as pltpu
from jax.experimental.pallas import tpu_sc as plsc

_NUM_ITEMS = 100000
_EMBED_DIM = 64
_FROZEN_DIM = 128
_NUM_TAGS = 1000
_TAG_DIM = 16
_TAGS_PER_ITEM = 5
_TAG_PACK = 2

_NC = 2
_NS = 16
_NW = _NC * _NS

_B = 4096
_H = 50
_C = 128
_NBT = _B // _C
_NSLAB = _H * _NBT
_PER_W = _NSLAB // _NW
_NSTEP = _PER_W // 2
_OUT_DIM = _FROZEN_DIM + _EMBED_DIM + _TAG_DIM
_NDT = _OUT_DIM // 8
_PITCH = _C + 1
_L = 16


def _sc_body(ids_hbm, ids8_hbm, frozen_hbm, tags0_hbm, tags1_hbm, item_w_hbm,
             tag_w_hbm, w_hbm,
             idx0, idx1, idx2, idx3, idx8_0, idx8_1, idx8_2, idx8_3,
             frozen0, frozen1, learn0, learn1,
             tags0a, tags0b, tags1a, tags1b, slab0, slab1, tag_tab,
             gsem0, gsem1, wsem0, wsem1, isem0, isem1, isem2, isem3):
    wid = lax.axis_index("s") * _NC + lax.axis_index("c")
    base = wid * _PER_W

    idx_c = (idx0, idx1, idx2, idx3)
    idx8_c = (idx8_0, idx8_1, idx8_2, idx8_3)
    frozen_c = (frozen0, frozen1)
    learn_c = (learn0, learn1)
    tags0_c = (tags0a, tags0b)
    tags1_c = (tags1a, tags1b)
    slab_c = (slab0, slab1)
    gsem = (gsem0, gsem1)
    wsem = (wsem0, wsem1)
    isem = (isem0, isem1, isem2, isem3)

    pltpu.sync_copy(tag_w_hbm, tag_tab)

    iota = lax.iota(jnp.int32, _L)
    inv_tags = jnp.float32(1.0 / _TAGS_PER_ITEM)

    dtv = []
    drv = []
    for k in range(12):
        d = k * _L + iota
        dtv.append(lax.shift_right_logical(d, 3))
        drv.append(lax.bitwise_and(d, 7))

    def idx_start(g, r):
        pltpu.async_copy(ids_hbm.at[base + g], idx_c[r], isem[r])
        pltpu.async_copy(ids8_hbm.at[base + g], idx8_c[r], isem[r])

    def ig(g, r, b):
        pltpu.make_async_copy(ids_hbm.at[base], idx_c[r], isem[r]).wait()
        pltpu.make_async_copy(ids8_hbm.at[base], idx8_c[r], isem[r]).wait()
        idxr = idx_c[r]
        pltpu.async_copy(frozen_hbm.at[idxr], frozen_c[b], gsem[b])
        pltpu.async_copy(item_w_hbm.at[idxr], learn_c[b], gsem[b])
        pltpu.async_copy(tags0_hbm.at[idx8_c[r]], tags0_c[b], gsem[b])
        pltpu.async_copy(tags1_hbm.at[idx8_c[r]], tags1_c[b], gsem[b])

    def waitg(r, b):
        idxr = idx_c[r]
        pltpu.make_async_copy(frozen_hbm.at[idxr], frozen_c[b], gsem[b]).wait()
        pltpu.make_async_copy(item_w_hbm.at[idxr], learn_c[b], gsem[b]).wait()
        pltpu.make_async_copy(tags0_hbm.at[idx8_c[r]], tags0_c[b], gsem[b]).wait()
        pltpu.make_async_copy(tags1_hbm.at[idx8_c[r]], tags1_c[b], gsem[b]).wait()

    def iw(g, b):
        s = base + g
        l = s // _NBT
        bt = s % _NBT
        pltpu.async_copy(slab_c[b].at[:, :, pl.ds(0, _C)],
                         w_hbm.at[l, :, bt], wsem[b])

    def waitw(b):
        pltpu.make_async_copy(slab_c[b].at[:, :, pl.ds(0, _C)],
                              w_hbm.at[0, :, 0], wsem[b]).wait()

    def compute(r, b):
        fz, ln, W = frozen_c[b], learn_c[b], slab_c[b]
        tg0, tg1 = tags0_c[b], tags1_c[b]
        ids_v = idx_c[r]

        @plsc.parallel_loop(0, _C, 1, unroll=4)
        def _item(br):
            col = jnp.full((_L,), 0, jnp.int32) + br
            for k in range(8):
                v = fz[br, pl.ds(k * _L, _L)]
                plsc.store_scatter(W, [dtv[k], drv[k], col], v)
            for k in range(4):
                v = ln[br, pl.ds(k * _L, _L)]
                plsc.store_scatter(W, [dtv[8 + k], drv[8 + k], col], v)

        @plsc.parallel_loop(0, _C // _L, 1, unroll=2)
        def _grp(h):
            rows = h * _L + iota
            ids16 = ids_v[pl.ds(h * _L, _L)]
            col = lax.bitwise_and(ids16, 7)
            w0 = plsc.load_gather(tg0, [rows, col])
            w1 = plsc.load_gather(tg1, [rows, col])
            mask10 = jnp.full((_L,), 1023, jnp.int32)
            tv = [
                lax.bitwise_and(w0, mask10),
                lax.bitwise_and(lax.shift_right_logical(w0, 10), mask10),
                lax.shift_right_logical(w0, 20),
                lax.bitwise_and(w1, mask10),
                lax.shift_right_logical(w1, 10),
            ]
            for d in range(_TAG_DIM):
                dcol = jnp.full((_L,), d, jnp.int32)
                acc = plsc.load_gather(tag_tab, [tv[0], dcol])
                for t in range(1, _TAGS_PER_ITEM):
                    acc = acc + plsc.load_gather(tag_tab, [tv[t], dcol])
                dd = _FROZEN_DIM + _EMBED_DIM + d
                W[dd >> 3, dd & 7, pl.ds(h * _L, _L)] = acc * inv_tags

    idx_start(0, 0)
    idx_start(1, 1)
    ig(0, 0, 0)

    def step(s, carry):
        g0 = 2 * s
        ig(g0 + 1, 1, 1)
        waitg(0, 0)
        pl.when(s > 0)(lambda: waitw(0))
        compute(0, 0)
        pl.when(s < _NSTEP - 1)(lambda: idx_start(g0 + 2, 0))
        iw(g0, 0)
        pl.when(s < _NSTEP - 1)(lambda: ig(g0 + 2, 0, 0))
        waitg(1, 1)
        pl.when(s > 0)(lambda: waitw(1))
        compute(1, 1)
        pl.when(s < _NSTEP - 1)(lambda: idx_start(g0 + 3, 1))
        iw(g0 + 1, 1)
        return carry

    lax.fori_loop(0, _NSTEP, step, 0)
    waitw(0)
    waitw(1)


_sc_kernel = functools.partial(
    pl.kernel,
    out_type=jax.ShapeDtypeStruct((_H, _NDT, _NBT, 8, _C), jnp.float32),
    mesh=plsc.VectorSubcoreMesh(core_axis_name="c", subcore_axis_name="s",
                                num_cores=_NC, num_subcores=_NS),
    scratch_types=[
        pltpu.VMEM((_C,), jnp.int32),
        pltpu.VMEM((_C,), jnp.int32),
        pltpu.VMEM((_C,), jnp.int32),
        pltpu.VMEM((_C,), jnp.int32),
        pltpu.VMEM((_C,), jnp.int32),
        pltpu.VMEM((_C,), jnp.int32),
        pltpu.VMEM((_C,), jnp.int32),
        pltpu.VMEM((_C,), jnp.int32),
        pltpu.VMEM((_C, _FROZEN_DIM), jnp.float32),
        pltpu.VMEM((_C, _FROZEN_DIM), jnp.float32),
        pltpu.VMEM((_C, _EMBED_DIM), jnp.float32),
        pltpu.VMEM((_C, _EMBED_DIM), jnp.float32),
        pltpu.VMEM((_C, 8), jnp.int32),
        pltpu.VMEM((_C, 8), jnp.int32),
        pltpu.VMEM((_C, 8), jnp.int32),
        pltpu.VMEM((_C, 8), jnp.int32),
        pltpu.VMEM((_NDT, 8, _PITCH), jnp.float32),
        pltpu.VMEM((_NDT, 8, _PITCH), jnp.float32),
        pltpu.VMEM((_NUM_TAGS, _TAG_DIM), jnp.float32),
        pltpu.SemaphoreType.DMA,
        pltpu.SemaphoreType.DMA,
        pltpu.SemaphoreType.DMA,
        pltpu.SemaphoreType.DMA,
        pltpu.SemaphoreType.DMA,
        pltpu.SemaphoreType.DMA,
        pltpu.SemaphoreType.DMA,
        pltpu.SemaphoreType.DMA,
    ],
    compiler_params=pltpu.CompilerParams(use_tc_tiling_on_sc=False,
                                         needs_layout_passes=False),
)(_sc_body)


def kernel(item_ids, frozen_emb, item_tags, item_emb_w, tag_emb_w):
    ids_t = item_ids.astype(jnp.int32).T.reshape(_NSLAB, _C)
    t = item_tags.astype(jnp.int32)
    w0 = t[:, 0] | (t[:, 1] << 10) | (t[:, 2] << 20)
    w1 = t[:, 3] | (t[:, 4] << 10)
    w0t = w0.reshape(_NUM_ITEMS // 8, 8)
    w1t = w1.reshape(_NUM_ITEMS // 8, 8)
    ids8 = ids_t >> 3
    w = _sc_kernel(ids_t, ids8, frozen_emb, w0t, w1t, item_emb_w, tag_emb_w)
    return jnp.transpose(w, (2, 4, 0, 1, 3)).reshape(_B, _H, _OUT_DIM)
